# baseline (speedup 1.0000x reference)
import functools

import jax
import jax.numpy as jnp
from jax import lax
from jax.experimental import pallas as pl
from jax.experimental.pallas import tpu as pltpu
from jax.experimental.pallas import tpu_sc as plsc

NTOKEN = 1000000
NINP = 64

C = 128
NBUF = 8


def _sc_gather(idx2d, table):
    nchunks, c = idx2d.shape
    v, d = table.shape
    info = plsc.get_sparse_core_info()
    nw = info.num_cores * info.num_subcores
    cpw = nchunks // nw
    n_outer = cpw // NBUF

    mesh = plsc.VectorSubcoreMesh(core_axis_name="c", subcore_axis_name="s")

    @functools.partial(
        pl.kernel,
        mesh=mesh,
        out_type=jax.ShapeDtypeStruct((nchunks * c, d), jnp.float32),
        compiler_params=pltpu.CompilerParams(use_tc_tiling_on_sc=False),
        scratch_types=[
            pltpu.VMEM((cpw, c), jnp.int32),
            pltpu.VMEM((NBUF, c, d), jnp.float32),
            pltpu.SemaphoreType.DMA((NBUF,)),
        ],
    )
    def body(idx_hbm, table_hbm, out_hbm, idx_v, rows_v, gsem):
        wid = lax.axis_index("s") * info.num_cores + lax.axis_index("c")
        chunk0 = wid * cpw
        row0 = chunk0 * c

        pltpu.sync_copy(idx_hbm.at[pl.ds(chunk0, cpw)], idx_v)

        for b in range(NBUF):
            pltpu.make_async_copy(
                table_hbm.at[idx_v.at[b]], rows_v.at[b], gsem.at[b]
            ).start()

        def outer(g0, carry):
            for b in range(NBUF):
                j = g0 * NBUF + b
                pltpu.make_async_copy(
                    table_hbm.at[idx_v.at[j]], rows_v.at[b], gsem.at[b]
                ).wait()
                pltpu.sync_copy(
                    rows_v.at[b], out_hbm.at[pl.ds(row0 + j * c, c)]
                )
                @pl.when(g0 < n_outer - 1)
                def _():
                    pltpu.make_async_copy(
                        table_hbm.at[idx_v.at[j + NBUF]], rows_v.at[b],
                        gsem.at[b],
                    ).start()
            return carry

        lax.fori_loop(0, n_outer, outer, 0)

    return body(idx2d, table)


def kernel(input, encoder_weight):
    b0, s = input.shape
    flat = input.reshape(-1).astype(jnp.int32)
    idx2d = flat.reshape(-1, C)
    out = _sc_gather(idx2d, encoder_weight)
    return out.reshape(b0, s, NINP)


# speedup vs baseline: 1.0022x; 1.0022x over previous
import functools

import jax
import jax.numpy as jnp
from jax import lax
from jax.experimental import pallas as pl
from jax.experimental.pallas import tpu as pltpu
from jax.experimental.pallas import tpu_sc as plsc

NTOKEN = 1000000
NINP = 64

C = 128
NBUF = 10
G = 8


def _sc_gather(idx2d, table):
    nchunks, c = idx2d.shape
    v, d = table.shape
    info = plsc.get_sparse_core_info()
    nw = info.num_cores * info.num_subcores
    cpw = nchunks // nw
    n_outer = cpw // NBUF

    mesh = plsc.VectorSubcoreMesh(core_axis_name="c", subcore_axis_name="s")

    @functools.partial(
        pl.kernel,
        mesh=mesh,
        out_type=jax.ShapeDtypeStruct((nchunks * c, d), jnp.float32),
        compiler_params=pltpu.CompilerParams(use_tc_tiling_on_sc=False),
        scratch_types=[
            pltpu.VMEM((cpw, c), jnp.int32),
            pltpu.VMEM((NBUF, c, d), jnp.float32),
            pltpu.SemaphoreType.DMA((NBUF,)),
            pltpu.SemaphoreType.DMA((NBUF,)),
        ],
    )
    def body(idx_hbm, table_hbm, out_hbm, idx_v, rows_v, gsem, osem):
        wid = lax.axis_index("s") * info.num_cores + lax.axis_index("c")
        chunk0 = wid * cpw
        row0 = chunk0 * c

        pltpu.sync_copy(idx_hbm.at[pl.ds(chunk0, cpw)], idx_v)

        for k in range(G):
            pltpu.make_async_copy(
                table_hbm.at[idx_v.at[k]], rows_v.at[k], gsem.at[k]
            ).start()

        def outer(g0, carry):
            for b in range(NBUF):
                j = g0 * NBUF + b
                bn = (b + G) % NBUF
                pltpu.make_async_copy(
                    table_hbm.at[idx_v.at[j]], rows_v.at[b], gsem.at[b]
                ).wait()
                pltpu.make_async_copy(
                    rows_v.at[b], out_hbm.at[pl.ds(row0 + j * c, c)],
                    osem.at[b],
                ).start()
                @pl.when(jnp.logical_and(j + G < cpw, j + G >= NBUF))
                def _():
                    pltpu.make_async_copy(
                        rows_v.at[bn],
                        out_hbm.at[pl.ds(row0 + (j + G - NBUF) * c, c)],
                        osem.at[bn],
                    ).wait()

                @pl.when(j + G < cpw)
                def _():
                    pltpu.make_async_copy(
                        table_hbm.at[idx_v.at[j + G]], rows_v.at[bn],
                        gsem.at[bn],
                    ).start()
            return carry

        lax.fori_loop(0, n_outer, outer, 0)

        for b in range(NBUF):
            j = cpw - NBUF + b
            pltpu.make_async_copy(
                rows_v.at[b], out_hbm.at[pl.ds(row0 + j * c, c)], osem.at[b]
            ).wait()

    return body(idx2d, table)


def kernel(input, encoder_weight):
    b0, s = input.shape
    flat = input.reshape(-1).astype(jnp.int32)
    idx2d = flat.reshape(-1, C)
    out = _sc_gather(idx2d, encoder_weight)
    return out.reshape(b0, s, NINP)
